# baseline (speedup 1.0000x reference)
import jax
import jax.numpy as jnp
from jax.experimental import pallas as pl

_K = 8
_ITERS = 10
_B = 4


def _dot(a, b, dims):
    return jax.lax.dot_general(a, b, (dims, ((), ())),
                               preferred_element_type=jnp.float32)


def _split3(a):
    h = a.astype(jnp.bfloat16)
    l = (a - h.astype(jnp.float32)).astype(jnp.bfloat16)
    l2 = (a - h.astype(jnp.float32) - l.astype(jnp.float32)).astype(jnp.bfloat16)
    return h, l, l2


def _argmin_k(d):
    best = d[0:1, :]
    idx = jnp.zeros_like(best, dtype=jnp.int32)
    for k in range(1, _K):
        row = d[k:k + 1, :]
        m = row < best
        best = jnp.where(m, row, best)
        idx = jnp.where(m, k, idx)
    return idx


def _kmeans_kernel(x_ref, labels_ref, clustered_ref):
    C = x_ref.shape[1]
    N = x_ref.shape[2]
    BK, BC = _B * _K, _B * C

    x_stack = jnp.concatenate([x_ref[b] for b in range(_B)], axis=0)
    xh, xl, xl2 = _split3(x_stack)

    row_b = jax.lax.broadcasted_iota(jnp.int32, (BK, BC), 0) // _K
    col_b = jax.lax.broadcasted_iota(jnp.int32, (BK, BC), 1) // C
    diag = row_b == col_b

    sel_i = jax.lax.broadcasted_iota(jnp.int32, (_K, N), 0)
    sel_n = jax.lax.broadcasted_iota(jnp.int32, (_K, N), 1)
    sel = (sel_i == sel_n).astype(jnp.bfloat16)
    dims_n = ((1,), (1,))
    c0 = (_dot(sel, xh, dims_n) + _dot(sel, xl, dims_n)
          + _dot(sel, xl2, dims_n))
    c_all = jnp.where(diag, jnp.concatenate([c0] * _B, axis=0), 0.0)

    iota_k = jax.lax.broadcasted_iota(jnp.int32, (_K, N), 0)
    dims_c = ((1,), (0,))

    def step(c_all):
        csq = jnp.sum(c_all * c_all, axis=1, keepdims=True)
        cx = jax.lax.dot_general(
            c_all, x_stack, (dims_c, ((), ())),
            preferred_element_type=jnp.float32,
            precision=jax.lax.Precision.HIGHEST)
        d_all = csq - 2.0 * cx
        labs = [_argmin_k(d_all[b * _K:(b + 1) * _K, :]) for b in range(_B)]
        ohm = jnp.concatenate([labs[b] == iota_k for b in range(_B)], axis=0)
        oh = ohm.astype(jnp.bfloat16)
        sums = (_dot(oh, xh, dims_n) + _dot(oh, xl, dims_n)
                + _dot(oh, xl2, dims_n))
        counts = jnp.sum(ohm.astype(jnp.float32), axis=1, keepdims=True)
        return labs, sums, counts

    for _ in range(_ITERS):
        _, sums, counts = step(c_all)
        c_all = jnp.where(jnp.logical_and(diag, counts > 0),
                          sums / jnp.maximum(counts, 1.0), c_all)

    labs, sums, counts = step(c_all)
    out = jnp.where(jnp.logical_and(diag, counts > 0),
                    sums / jnp.maximum(counts, 1.0), 0.0)
    for b in range(_B):
        clustered_ref[b] = out[b * _K:(b + 1) * _K, b * C:(b + 1) * C]
        labels_ref[b] = labs[b]


def kernel(feature_map):
    B, C, H, W = feature_map.shape
    N = H * W
    x = feature_map.reshape(B, C, N)
    labels3, clustered = pl.pallas_call(
        _kmeans_kernel,
        out_shape=[
            jax.ShapeDtypeStruct((B, 1, N), jnp.int32),
            jax.ShapeDtypeStruct((B, _K, C), jnp.float32),
        ],
    )(x)
    return clustered, labels3.reshape(B, N)


# speedup vs baseline: 1.1806x; 1.1806x over previous
import jax
import jax.numpy as jnp
from jax.experimental import pallas as pl

_K = 8
_ITERS = 10
_B = 4


def _dot(a, b, dims):
    return jax.lax.dot_general(a, b, (dims, ((), ())),
                               preferred_element_type=jnp.float32)


def _split3(a):
    h = a.astype(jnp.bfloat16)
    l = (a - h.astype(jnp.float32)).astype(jnp.bfloat16)
    l2 = (a - h.astype(jnp.float32) - l.astype(jnp.float32)).astype(jnp.bfloat16)
    return h, l, l2


def _dists(c, xcat, C):
    ch = c.astype(jnp.bfloat16)
    cl = (c - ch.astype(jnp.float32)).astype(jnp.bfloat16)
    cl2 = (c - ch.astype(jnp.float32) - cl.astype(jnp.float32)).astype(jnp.bfloat16)
    ccat = jnp.concatenate([ch, ch, ch, cl, cl, cl2], axis=1)
    cx = _dot(ccat, xcat, ((1,), (0,)))
    csq = jnp.sum(c * c, axis=1, keepdims=True)
    return csq - 2.0 * cx


def _argmin_k(d):
    best = d[0:1, :]
    idx = jnp.zeros_like(best, dtype=jnp.int32)
    for k in range(1, _K):
        row = d[k:k + 1, :]
        m = row < best
        best = jnp.where(m, row, best)
        idx = jnp.where(m, k, idx)
    return idx


def _segment(lab, xcat, C):
    iota = jax.lax.broadcasted_iota(jnp.int32, (_K, lab.shape[1]), 0)
    ohm = lab == iota
    oh = ohm.astype(jnp.bfloat16)
    s3 = _dot(oh, xcat[:3 * C, :], ((1,), (1,)))
    sums = s3[:, :C] + s3[:, C:2 * C] + s3[:, 2 * C:3 * C]
    counts = jnp.sum(ohm.astype(jnp.float32), axis=1, keepdims=True)
    return sums, counts


def _kmeans_kernel(x_ref, labels_ref, clustered_ref):
    C = x_ref.shape[1]
    N = x_ref.shape[2]

    def build_stack(b):
        xh, xl, xl2 = _split3(x_ref[b])
        return jnp.concatenate([xh, xl, xl2, xh, xl, xh], axis=0)

    xcats = [build_stack(b) for b in range(_B)]

    sel_i = jax.lax.broadcasted_iota(jnp.int32, (_K, N), 0)
    sel_n = jax.lax.broadcasted_iota(jnp.int32, (_K, N), 1)
    sel = (sel_i == sel_n).astype(jnp.bfloat16)
    cs = []
    for b in range(_B):
        s3 = _dot(sel, xcats[b][:3 * C, :], ((1,), (1,)))
        cs.append(s3[:, :C] + s3[:, C:2 * C] + s3[:, 2 * C:3 * C])

    for _ in range(_ITERS):
        labs = [_argmin_k(_dists(cs[b], xcats[b], C)) for b in range(_B)]
        for b in range(_B):
            sums, counts = _segment(labs[b], xcats[b], C)
            cs[b] = jnp.where(counts > 0, sums / jnp.maximum(counts, 1.0), cs[b])

    for b in range(_B):
        lab = _argmin_k(_dists(cs[b], xcats[b], C))
        sums, counts = _segment(lab, xcats[b], C)
        clustered_ref[b] = jnp.where(counts > 0, sums / jnp.maximum(counts, 1.0), 0.0)
        labels_ref[b] = lab


def kernel(feature_map):
    B, C, H, W = feature_map.shape
    N = H * W
    x = feature_map.reshape(B, C, N)
    labels3, clustered = pl.pallas_call(
        _kmeans_kernel,
        out_shape=[
            jax.ShapeDtypeStruct((B, 1, N), jnp.int32),
            jax.ShapeDtypeStruct((B, _K, C), jnp.float32),
        ],
    )(x)
    return clustered, labels3.reshape(B, N)


# speedup vs baseline: 1.1819x; 1.0010x over previous
import jax
import jax.numpy as jnp
from jax.experimental import pallas as pl

_K = 8
_ITERS = 10
_B = 4


def _dot(a, b, dims):
    return jax.lax.dot_general(a, b, (dims, ((), ())),
                               preferred_element_type=jnp.float32)


def _split3(a):
    h = a.astype(jnp.bfloat16)
    l = (a - h.astype(jnp.float32)).astype(jnp.bfloat16)
    l2 = (a - h.astype(jnp.float32) - l.astype(jnp.float32)).astype(jnp.bfloat16)
    return h, l, l2


def _dists(c, xcat, C):
    cm2 = -2.0 * c
    ch = cm2.astype(jnp.bfloat16)
    cl = (cm2 - ch.astype(jnp.float32)).astype(jnp.bfloat16)
    cl2 = (cm2 - ch.astype(jnp.float32) - cl.astype(jnp.float32)).astype(jnp.bfloat16)
    ccat = jnp.concatenate([ch, ch, ch, cl, cl, cl2], axis=1)
    cx = _dot(ccat, xcat, ((1,), (0,)))
    csq = jnp.sum(c * c, axis=1, keepdims=True)
    return csq + cx


def _argmin_k(d):
    dmin = jnp.min(d, axis=0, keepdims=True)
    iota = jax.lax.broadcasted_iota(jnp.int32, d.shape, 0)
    cand = jnp.where(d == dmin, iota, _K)
    return jnp.min(cand, axis=0, keepdims=True)


def _segment(lab, xcat, C):
    iota = jax.lax.broadcasted_iota(jnp.int32, (_K, lab.shape[1]), 0)
    ohm = lab == iota
    oh = ohm.astype(jnp.bfloat16)
    s3 = _dot(oh, xcat[:3 * C, :], ((1,), (1,)))
    sums = s3[:, :C] + s3[:, C:2 * C] + s3[:, 2 * C:3 * C]
    counts = jnp.sum(ohm.astype(jnp.float32), axis=1, keepdims=True)
    return sums, counts


def _kmeans_kernel(x_ref, labels_ref, clustered_ref):
    C = x_ref.shape[1]
    N = x_ref.shape[2]

    def build_stack(b):
        xh, xl, xl2 = _split3(x_ref[b])
        return jnp.concatenate([xh, xl, xl2, xh, xl, xh], axis=0)

    xcats = [build_stack(b) for b in range(_B)]

    sel_i = jax.lax.broadcasted_iota(jnp.int32, (_K, N), 0)
    sel_n = jax.lax.broadcasted_iota(jnp.int32, (_K, N), 1)
    sel = (sel_i == sel_n).astype(jnp.bfloat16)
    cs = []
    for b in range(_B):
        s3 = _dot(sel, xcats[b][:3 * C, :], ((1,), (1,)))
        cs.append(s3[:, :C] + s3[:, C:2 * C] + s3[:, 2 * C:3 * C])

    for _ in range(_ITERS):
        labs = [_argmin_k(_dists(cs[b], xcats[b], C)) for b in range(_B)]
        for b in range(_B):
            sums, counts = _segment(labs[b], xcats[b], C)
            cs[b] = jnp.where(counts > 0, sums / jnp.maximum(counts, 1.0), cs[b])

    for b in range(_B):
        lab = _argmin_k(_dists(cs[b], xcats[b], C))
        sums, counts = _segment(lab, xcats[b], C)
        clustered_ref[b] = jnp.where(counts > 0, sums / jnp.maximum(counts, 1.0), 0.0)
        labels_ref[b] = lab


def kernel(feature_map):
    B, C, H, W = feature_map.shape
    N = H * W
    x = feature_map.reshape(B, C, N)
    labels3, clustered = pl.pallas_call(
        _kmeans_kernel,
        out_shape=[
            jax.ShapeDtypeStruct((B, 1, N), jnp.int32),
            jax.ShapeDtypeStruct((B, _K, C), jnp.float32),
        ],
    )(x)
    return clustered, labels3.reshape(B, N)
